# baseline (speedup 1.0000x reference)
import jax
import jax.numpy as jnp
from jax import lax
from jax.experimental import pallas as pl
from jax.experimental.pallas import tpu as pltpu
from jax.experimental.pallas import tpu_sc as plsc

VOCAB = 100000
EMB = 64
REGION = 3
B, L = 1024, 200
TROW = (1 + REGION) * EMB

NC, NS = 2, 16
NW = NC * NS
RPW = B // NW
LP = 208
SPLIT = 112
NCHUNK = LP // 16


def _load_seq_and_indices(seq_hbm, row, seq_v, idx_a, idx_b):
    pltpu.sync_copy(seq_hbm.at[pl.ds(row * L, L)], seq_v.at[pl.ds(0, L)])
    for k in range(NCHUNK):
        s = seq_v[pl.ds(k * 16, 16)]
        idx = jnp.maximum(s, 1) - 1
        if k * 16 < SPLIT:
            idx_a[pl.ds(k * 16, 16)] = idx
        else:
            idx_b[pl.ds(k * 16 - SPLIT, 16)] = idx


def _compute_masks(seq_v, m_v):
    for k in range(NCHUNK):
        s = seq_v[pl.ds(k * 16, 16)]
        m_v[pl.ds(k * 16 + 16, 16)] = jnp.where(
            s != 0, jnp.int32(-1), jnp.int32(0))


def _fire_gathers(t_hbm, idx_a, idx_b, t_rows, sem):
    pltpu.async_copy(t_hbm.at[idx_a], t_rows.at[pl.ds(1, SPLIT)], sem)
    pltpu.async_copy(t_hbm.at[idx_b],
                     t_rows.at[pl.ds(1 + SPLIT, LP - SPLIT)], sem)


def _wait_gathers(t_hbm, idx_a, idx_b, t_rows, sem):
    pltpu.make_async_copy(
        t_hbm.at[idx_a], t_rows.at[pl.ds(1, SPLIT)], sem).wait()
    pltpu.make_async_copy(
        t_hbm.at[idx_b], t_rows.at[pl.ds(1 + SPLIT, LP - SPLIT)], sem).wait()


def _compute_row(t_rows, m_v, h_rows):
    @pl.loop(0, L, unroll=4)
    def _token(l):
        mm = m_v[pl.ds(l + 15, 16)]
        ml = mm[0]
        mc = mm[1]
        mr = mm[2]
        for c in range(EMB // 32):
            wr = t_rows[l + 1, pl.ds(c * 32, 32)]
            w = plsc.bitcast(plsc.bitcast(wr, jnp.int32) & mc, jnp.bfloat16)
            p0 = t_rows[l, pl.ds(EMB + c * 32, 32)] * w
            p0 = plsc.bitcast(plsc.bitcast(p0, jnp.int32) & ml, jnp.bfloat16)
            p1 = t_rows[l + 1, pl.ds(2 * EMB + c * 32, 32)] * w
            p2 = t_rows[l + 2, pl.ds(3 * EMB + c * 32, 32)] * w
            p2 = plsc.bitcast(plsc.bitcast(p2, jnp.int32) & mr, jnp.bfloat16)
            h_rows[pl.ds(l * EMB + c * 32, 32)] = jnp.maximum(
                jnp.maximum(p0, p1), p2)


def _region_kernel(seq_hbm, t_hbm, out_hbm,
                   seq_v, m_v,
                   idx_a0, idx_b0, t_rows0,
                   idx_a1, idx_b1, t_rows1,
                   h_rows, sem0, sem1, sem_out):
    wid = lax.axis_index("s") * NC + lax.axis_index("c")
    base = wid * RPW

    zi = jnp.zeros((16,), jnp.int32)
    zf = jnp.zeros((16,), jnp.float32)
    zb = jnp.zeros((32,), jnp.bfloat16)
    seq_v[pl.ds(192, 16)] = zi
    m_v[pl.ds(0, 16)] = zi
    for c in range(TROW // 32):
        t_rows0[0, pl.ds(c * 32, 32)] = zb
        t_rows1[0, pl.ds(c * 32, 32)] = zb

    bufs = ((idx_a0, idx_b0, t_rows0, sem0),
            (idx_a1, idx_b1, t_rows1, sem1))

    _load_seq_and_indices(seq_hbm, base, seq_v, idx_a0, idx_b0)
    _fire_gathers(t_hbm, idx_a0, idx_b0, t_rows0, sem0)

    @pl.loop(0, RPW, step=2)
    def _pair(j):
        for b in range(2):
            r = j + b
            row = base + r
            ia, ib, tr, sem = bufs[b]
            ia2, ib2, tr2, sem2 = bufs[1 - b]

            _compute_masks(seq_v, m_v)

            @pl.when(r + 1 < RPW)
            def _prefetch():
                _load_seq_and_indices(seq_hbm, row + 1, seq_v, ia2, ib2)
                _fire_gathers(t_hbm, ia2, ib2, tr2, sem2)

            _wait_gathers(t_hbm, ia, ib, tr, sem)

            @pl.when(r > 0)
            def _drain_prev_out():
                pltpu.make_async_copy(
                    h_rows, out_hbm.at[pl.ds((row - 1) * L * EMB, L * EMB)],
                    sem_out).wait()

            _compute_row(tr, m_v, h_rows)
            pltpu.async_copy(
                h_rows, out_hbm.at[pl.ds(row * L * EMB, L * EMB)], sem_out)

    pltpu.make_async_copy(
        h_rows, out_hbm.at[pl.ds((base + RPW - 1) * L * EMB, L * EMB)],
        sem_out).wait()


@jax.jit
def _run(seq, T):
    mesh = plsc.VectorSubcoreMesh(
        core_axis_name="c", subcore_axis_name="s",
        num_cores=NC, num_subcores=NS)
    dbuf = [
        pltpu.VMEM((SPLIT,), jnp.int32),
        pltpu.VMEM((LP - SPLIT,), jnp.int32),
        pltpu.VMEM((LP + 1, TROW), jnp.bfloat16),
    ]
    kfn = pl.kernel(
        _region_kernel,
        out_type=jax.ShapeDtypeStruct((B * L * EMB,), jnp.bfloat16),
        mesh=mesh,
        compiler_params=pltpu.CompilerParams(
            use_tc_tiling_on_sc=False, needs_layout_passes=False),
        scratch_types=[
            pltpu.VMEM((LP,), jnp.int32),
            pltpu.VMEM((LP + 32,), jnp.int32),
            *dbuf, *dbuf,
            pltpu.VMEM((L * EMB,), jnp.bfloat16),
            pltpu.SemaphoreType.DMA,
            pltpu.SemaphoreType.DMA,
            pltpu.SemaphoreType.DMA,
        ],
    )
    return kfn(seq, T)


def kernel(seq, W, U):
    seq = seq.astype(jnp.int32).reshape(B * L)
    eps1 = jnp.float32(1.0 + 2.0 ** -23)
    T = (jnp.concatenate(
        [W, U.reshape(VOCAB - 1, REGION * EMB)], axis=1)
        * eps1).astype(jnp.bfloat16)
    h = _run(seq, T).astype(jnp.float32) * eps1
    return h.reshape(B, L, EMB)


# speedup vs baseline: 1.0907x; 1.0907x over previous
import jax
import jax.numpy as jnp
from jax import lax
from jax.experimental import pallas as pl
from jax.experimental.pallas import tpu as pltpu
from jax.experimental.pallas import tpu_sc as plsc

VOCAB = 100000
EMB = 64
REGION = 3
B, L = 1024, 200
UROW = REGION * EMB

NC, NS = 2, 16
NW = NC * NS
RPW = B // NW
LP = 208
SPLIT = 112
NCHUNK = LP // 16


def _load_seq_and_indices(seq_hbm, row, seq_v, idx_a, idx_b):
    pltpu.sync_copy(seq_hbm.at[pl.ds(row * L, L)], seq_v.at[pl.ds(0, L)])
    for k in range(NCHUNK):
        s = seq_v[pl.ds(k * 16, 16)]
        idx = jnp.maximum(s, 1) - 1
        if k * 16 < SPLIT:
            idx_a[pl.ds(k * 16, 16)] = idx
        else:
            idx_b[pl.ds(k * 16 - SPLIT, 16)] = idx


def _compute_masks(seq_v, m_v):
    for k in range(NCHUNK):
        s = seq_v[pl.ds(k * 16, 16)]
        m_v[pl.ds(k * 16 + 16, 16)] = jnp.where(
            s != 0, jnp.int32(-1), jnp.int32(0))


def _fire_gathers(w_hbm, u3_hbm, idx_a, idx_b, w_rows, u_rows, sem):
    pltpu.async_copy(w_hbm.at[idx_a], w_rows.at[pl.ds(0, SPLIT)], sem)
    pltpu.async_copy(w_hbm.at[idx_b],
                     w_rows.at[pl.ds(SPLIT, LP - SPLIT)], sem)
    pltpu.async_copy(u3_hbm.at[idx_a], u_rows.at[pl.ds(1, SPLIT)], sem)
    pltpu.async_copy(u3_hbm.at[idx_b],
                     u_rows.at[pl.ds(1 + SPLIT, LP - SPLIT)], sem)


def _wait_gathers(w_hbm, u3_hbm, idx_a, idx_b, w_rows, u_rows, sem):
    pltpu.make_async_copy(
        w_hbm.at[idx_a], w_rows.at[pl.ds(0, SPLIT)], sem).wait()
    pltpu.make_async_copy(
        w_hbm.at[idx_b], w_rows.at[pl.ds(SPLIT, LP - SPLIT)], sem).wait()
    pltpu.make_async_copy(
        u3_hbm.at[idx_a], u_rows.at[pl.ds(1, SPLIT)], sem).wait()
    pltpu.make_async_copy(
        u3_hbm.at[idx_b], u_rows.at[pl.ds(1 + SPLIT, LP - SPLIT)], sem).wait()


def _compute_row(w_rows, u_rows, m_v, h_rows):
    @pl.loop(0, L, unroll=4)
    def _token(l):
        mm = m_v[pl.ds(l + 15, 16)]
        ml = mm[0]
        mc = mm[1]
        mr = mm[2]
        for c in range(EMB // 32):
            wr = w_rows[l, pl.ds(c * 32, 32)]
            w = plsc.bitcast(plsc.bitcast(wr, jnp.int32) & mc, jnp.bfloat16)
            p0 = u_rows[l, pl.ds(c * 32, 32)] * w
            p0 = plsc.bitcast(plsc.bitcast(p0, jnp.int32) & ml, jnp.bfloat16)
            p1 = u_rows[l + 1, pl.ds(EMB + c * 32, 32)] * w
            p2 = u_rows[l + 2, pl.ds(2 * EMB + c * 32, 32)] * w
            p2 = plsc.bitcast(plsc.bitcast(p2, jnp.int32) & mr, jnp.bfloat16)
            h_rows[pl.ds(l * EMB + c * 32, 32)] = jnp.maximum(
                jnp.maximum(p0, p1), p2)


def _region_kernel(seq_hbm, w_hbm, u3_hbm, out_hbm,
                   seq_v, m_v,
                   idx_a0, idx_b0, w_rows0, u_rows0,
                   idx_a1, idx_b1, w_rows1, u_rows1,
                   h_rows, sem0, sem1, sem_out):
    wid = lax.axis_index("s") * NC + lax.axis_index("c")
    base = wid * RPW

    zi = jnp.zeros((16,), jnp.int32)
    zb = jnp.zeros((32,), jnp.bfloat16)
    seq_v[pl.ds(192, 16)] = zi
    m_v[pl.ds(0, 16)] = zi
    for c in range(UROW // 32):
        u_rows0[0, pl.ds(c * 32, 32)] = zb
        u_rows1[0, pl.ds(c * 32, 32)] = zb

    bufs = ((idx_a0, idx_b0, w_rows0, u_rows0, sem0),
            (idx_a1, idx_b1, w_rows1, u_rows1, sem1))

    _load_seq_and_indices(seq_hbm, base, seq_v, idx_a0, idx_b0)
    _fire_gathers(w_hbm, u3_hbm, idx_a0, idx_b0, w_rows0, u_rows0, sem0)

    @pl.loop(0, RPW, step=2)
    def _pair(j):
        for b in range(2):
            r = j + b
            row = base + r
            ia, ib, wr_, ur, sem = bufs[b]
            ia2, ib2, wr2, ur2, sem2 = bufs[1 - b]

            _compute_masks(seq_v, m_v)

            @pl.when(r + 1 < RPW)
            def _prefetch():
                _load_seq_and_indices(seq_hbm, row + 1, seq_v, ia2, ib2)
                _fire_gathers(w_hbm, u3_hbm, ia2, ib2, wr2, ur2, sem2)

            _wait_gathers(w_hbm, u3_hbm, ia, ib, wr_, ur, sem)

            @pl.when(r > 0)
            def _drain_prev_out():
                pltpu.make_async_copy(
                    h_rows, out_hbm.at[pl.ds((row - 1) * L * EMB, L * EMB)],
                    sem_out).wait()

            _compute_row(wr_, ur, m_v, h_rows)
            pltpu.async_copy(
                h_rows, out_hbm.at[pl.ds(row * L * EMB, L * EMB)], sem_out)

    pltpu.make_async_copy(
        h_rows, out_hbm.at[pl.ds((base + RPW - 1) * L * EMB, L * EMB)],
        sem_out).wait()


@jax.jit
def _run(seq, Wb, U3b):
    mesh = plsc.VectorSubcoreMesh(
        core_axis_name="c", subcore_axis_name="s",
        num_cores=NC, num_subcores=NS)
    dbuf = [
        pltpu.VMEM((SPLIT,), jnp.int32),
        pltpu.VMEM((LP - SPLIT,), jnp.int32),
        pltpu.VMEM((LP, EMB), jnp.bfloat16),
        pltpu.VMEM((LP + 1, UROW), jnp.bfloat16),
    ]
    kfn = pl.kernel(
        _region_kernel,
        out_type=jax.ShapeDtypeStruct((B * L * EMB,), jnp.bfloat16),
        mesh=mesh,
        compiler_params=pltpu.CompilerParams(
            use_tc_tiling_on_sc=False, needs_layout_passes=False),
        scratch_types=[
            pltpu.VMEM((LP,), jnp.int32),
            pltpu.VMEM((LP + 32,), jnp.int32),
            *dbuf, *dbuf,
            pltpu.VMEM((L * EMB,), jnp.bfloat16),
            pltpu.SemaphoreType.DMA,
            pltpu.SemaphoreType.DMA,
            pltpu.SemaphoreType.DMA,
        ],
    )
    return kfn(seq, Wb, U3b)


def kernel(seq, W, U):
    seq = seq.astype(jnp.int32).reshape(B * L)
    Wb = W.astype(jnp.bfloat16)
    U3b = U.reshape(VOCAB - 1, REGION * EMB).astype(jnp.bfloat16)
    return _run(seq, Wb, U3b).astype(jnp.float32).reshape(B, L, EMB)


# speedup vs baseline: 1.1855x; 1.0870x over previous
import jax
import jax.numpy as jnp
from jax import lax
from jax.experimental import pallas as pl
from jax.experimental.pallas import tpu as pltpu
from jax.experimental.pallas import tpu_sc as plsc

VOCAB = 100000
EMB = 64
REGION = 3
B, L = 1024, 200
UROW = REGION * EMB

NC, NS = 2, 16
NW = NC * NS
RPW = B // NW
LP = 208
SPLIT = 112
NCHUNK = LP // 16


def _load_seq_and_indices(seq_hbm, row, seq_v, idx_a, idx_b):
    pltpu.sync_copy(seq_hbm.at[pl.ds(row * L, L)], seq_v.at[pl.ds(0, L)])
    for k in range(NCHUNK):
        s = seq_v[pl.ds(k * 16, 16)]
        idx = jnp.maximum(s, 1) - 1
        if k * 16 < SPLIT:
            idx_a[pl.ds(k * 16, 16)] = idx
        else:
            idx_b[pl.ds(k * 16 - SPLIT, 16)] = idx


def _compute_masks(seq_v, m_v):
    for k in range(NCHUNK):
        s = seq_v[pl.ds(k * 16, 16)]
        m_v[pl.ds(k * 16 + 16, 16)] = jnp.where(
            s != 0, jnp.float32(1.0), jnp.float32(0.0))


def _fire_gathers(w_hbm, u3_hbm, idx_a, idx_b, w_rows, u_rows, sem):
    pltpu.async_copy(w_hbm.at[idx_a], w_rows.at[pl.ds(0, SPLIT)], sem)
    pltpu.async_copy(w_hbm.at[idx_b],
                     w_rows.at[pl.ds(SPLIT, LP - SPLIT)], sem)
    pltpu.async_copy(u3_hbm.at[idx_a], u_rows.at[pl.ds(1, SPLIT)], sem)
    pltpu.async_copy(u3_hbm.at[idx_b],
                     u_rows.at[pl.ds(1 + SPLIT, LP - SPLIT)], sem)


def _wait_gathers(w_hbm, u3_hbm, idx_a, idx_b, w_rows, u_rows, sem):
    pltpu.make_async_copy(
        w_hbm.at[idx_a], w_rows.at[pl.ds(0, SPLIT)], sem).wait()
    pltpu.make_async_copy(
        w_hbm.at[idx_b], w_rows.at[pl.ds(SPLIT, LP - SPLIT)], sem).wait()
    pltpu.make_async_copy(
        u3_hbm.at[idx_a], u_rows.at[pl.ds(1, SPLIT)], sem).wait()
    pltpu.make_async_copy(
        u3_hbm.at[idx_b], u_rows.at[pl.ds(1 + SPLIT, LP - SPLIT)], sem).wait()


def _compute_row(w_rows, u_rows, m_v, h_rows):
    @pl.loop(0, L, unroll=4)
    def _token(l):
        mm = m_v[pl.ds(l + 15, 16)]
        ml = mm[0]
        mc = mm[1]
        mr = mm[2]
        for c in range(EMB // 16):
            w = w_rows[l, pl.ds(c * 16, 16)] * mc
            p0 = u_rows[l, pl.ds(c * 16, 16)] * w * ml
            p1 = u_rows[l + 1, pl.ds(EMB + c * 16, 16)] * w
            p2 = u_rows[l + 2, pl.ds(2 * EMB + c * 16, 16)] * w * mr
            h_rows[pl.ds(l * EMB + c * 16, 16)] = jnp.maximum(
                jnp.maximum(p0, p1), p2)


def _region_kernel(seq_hbm, w_hbm, u3_hbm, out_hbm,
                   seq_v, m_v,
                   idx_a0, idx_b0, w_rows0, u_rows0,
                   idx_a1, idx_b1, w_rows1, u_rows1,
                   h_rows, sem0, sem1, sem_out):
    wid = lax.axis_index("s") * NC + lax.axis_index("c")
    base = wid * RPW

    zi = jnp.zeros((16,), jnp.int32)
    zf = jnp.zeros((16,), jnp.float32)
    seq_v[pl.ds(192, 16)] = zi
    m_v[pl.ds(0, 16)] = zf
    for c in range(UROW // 16):
        u_rows0[0, pl.ds(c * 16, 16)] = zf
        u_rows1[0, pl.ds(c * 16, 16)] = zf

    bufs = ((idx_a0, idx_b0, w_rows0, u_rows0, sem0),
            (idx_a1, idx_b1, w_rows1, u_rows1, sem1))

    _load_seq_and_indices(seq_hbm, base, seq_v, idx_a0, idx_b0)
    _fire_gathers(w_hbm, u3_hbm, idx_a0, idx_b0, w_rows0, u_rows0, sem0)

    @pl.loop(0, RPW, step=2)
    def _pair(j):
        for b in range(2):
            r = j + b
            row = base + r
            ia, ib, wr_, ur, sem = bufs[b]
            ia2, ib2, wr2, ur2, sem2 = bufs[1 - b]

            _compute_masks(seq_v, m_v)

            @pl.when(r + 1 < RPW)
            def _prefetch():
                _load_seq_and_indices(seq_hbm, row + 1, seq_v, ia2, ib2)
                _fire_gathers(w_hbm, u3_hbm, ia2, ib2, wr2, ur2, sem2)

            _wait_gathers(w_hbm, u3_hbm, ia, ib, wr_, ur, sem)

            @pl.when(r > 0)
            def _drain_prev_out():
                pltpu.make_async_copy(
                    h_rows, out_hbm.at[pl.ds((row - 1) * L * EMB, L * EMB)],
                    sem_out).wait()

            _compute_row(wr_, ur, m_v, h_rows)
            pltpu.async_copy(
                h_rows, out_hbm.at[pl.ds(row * L * EMB, L * EMB)], sem_out)

    pltpu.make_async_copy(
        h_rows, out_hbm.at[pl.ds((base + RPW - 1) * L * EMB, L * EMB)],
        sem_out).wait()


@jax.jit
def _run(seq, W, U3):
    mesh = plsc.VectorSubcoreMesh(
        core_axis_name="c", subcore_axis_name="s",
        num_cores=NC, num_subcores=NS)
    dbuf = [
        pltpu.VMEM((SPLIT,), jnp.int32),
        pltpu.VMEM((LP - SPLIT,), jnp.int32),
        pltpu.VMEM((LP, EMB), jnp.float32),
        pltpu.VMEM((LP + 1, UROW), jnp.float32),
    ]
    kfn = pl.kernel(
        _region_kernel,
        out_type=jax.ShapeDtypeStruct((B * L * EMB,), jnp.float32),
        mesh=mesh,
        compiler_params=pltpu.CompilerParams(
            use_tc_tiling_on_sc=False, needs_layout_passes=False),
        scratch_types=[
            pltpu.VMEM((LP,), jnp.int32),
            pltpu.VMEM((LP + 32,), jnp.float32),
            *dbuf, *dbuf,
            pltpu.VMEM((L * EMB,), jnp.float32),
            pltpu.SemaphoreType.DMA,
            pltpu.SemaphoreType.DMA,
            pltpu.SemaphoreType.DMA,
        ],
    )
    return kfn(seq, W, U3)


def kernel(seq, W, U):
    seq = seq.astype(jnp.int32).reshape(B * L)
    U3 = U.reshape(VOCAB - 1, REGION * EMB)
    return _run(seq, W, U3).reshape(B, L, EMB)


# speedup vs baseline: 1.1871x; 1.0013x over previous
import jax
import jax.numpy as jnp
from jax import lax
from jax.experimental import pallas as pl
from jax.experimental.pallas import tpu as pltpu
from jax.experimental.pallas import tpu_sc as plsc

VOCAB = 100000
EMB = 64
REGION = 3
B, L = 1024, 200
UROW = REGION * EMB

NC, NS = 2, 16
NW = NC * NS
RPW = B // NW
LP = 208
SPLIT = 112
NCHUNK = LP // 16


def _load_seq_and_indices(seq_hbm, row, seq_v, idx_a, idx_b):
    pltpu.sync_copy(seq_hbm.at[pl.ds(row * L, L)], seq_v.at[pl.ds(0, L)])
    for k in range(NCHUNK):
        s = seq_v[pl.ds(k * 16, 16)]
        idx = jnp.maximum(s, 1) - 1
        if k * 16 < SPLIT:
            idx_a[pl.ds(k * 16, 16)] = idx
        else:
            idx_b[pl.ds(k * 16 - SPLIT, 16)] = idx


def _compute_masks(seq_v, m_v):
    for k in range(NCHUNK):
        s = seq_v[pl.ds(k * 16, 16)]
        m_v[pl.ds(k * 16 + 16, 16)] = jnp.where(
            s != 0, jnp.float32(1.0), jnp.float32(0.0))


def _fire_gathers(w_hbm, u3_hbm, idx_a, idx_b, w_rows, u_rows, sem):
    pltpu.async_copy(w_hbm.at[idx_a], w_rows.at[pl.ds(0, SPLIT)], sem)
    pltpu.async_copy(w_hbm.at[idx_b],
                     w_rows.at[pl.ds(SPLIT, LP - SPLIT)], sem)
    pltpu.async_copy(u3_hbm.at[idx_a], u_rows.at[pl.ds(1, SPLIT)], sem)
    pltpu.async_copy(u3_hbm.at[idx_b],
                     u_rows.at[pl.ds(1 + SPLIT, LP - SPLIT)], sem)


def _wait_gathers(w_hbm, u3_hbm, idx_a, idx_b, w_rows, u_rows, sem):
    pltpu.make_async_copy(
        w_hbm.at[idx_a], w_rows.at[pl.ds(0, SPLIT)], sem).wait()
    pltpu.make_async_copy(
        w_hbm.at[idx_b], w_rows.at[pl.ds(SPLIT, LP - SPLIT)], sem).wait()
    pltpu.make_async_copy(
        u3_hbm.at[idx_a], u_rows.at[pl.ds(1, SPLIT)], sem).wait()
    pltpu.make_async_copy(
        u3_hbm.at[idx_b], u_rows.at[pl.ds(1 + SPLIT, LP - SPLIT)], sem).wait()


def _compute_row(w_rows, u_rows, m_v, h_rows):
    @pl.loop(0, NCHUNK)
    def _group(g):
        ma = m_v[pl.ds(g * 16, 16)]
        mb = m_v[pl.ds(g * 16 + 16, 16)]
        mc_ = m_v[pl.ds(g * 16 + 32, 16)]
        gb = g * 16
        for t in range(16):
            ml = ma[15] if t == 0 else mb[t - 1]
            mm = mb[t]
            mr = mc_[0] if t == 15 else mb[t + 1]
            for c in range(EMB // 16):
                w = w_rows[gb + t, pl.ds(c * 16, 16)] * mm
                p0 = u_rows[gb + t, pl.ds(c * 16, 16)] * w * ml
                p1 = u_rows[gb + t + 1, pl.ds(EMB + c * 16, 16)] * w
                p2 = u_rows[gb + t + 2, pl.ds(2 * EMB + c * 16, 16)] * w * mr
                h_rows[pl.ds((gb + t) * EMB + c * 16, 16)] = jnp.maximum(
                    jnp.maximum(p0, p1), p2)


def _region_kernel(seq_hbm, w_hbm, u3_hbm, out_hbm,
                   seq_v, m_v,
                   idx_a0, idx_b0, w_rows0, u_rows0,
                   idx_a1, idx_b1, w_rows1, u_rows1,
                   h_rows, sem0, sem1, sem_out):
    wid = lax.axis_index("s") * NC + lax.axis_index("c")
    base = wid * RPW

    zi = jnp.zeros((16,), jnp.int32)
    zf = jnp.zeros((16,), jnp.float32)
    seq_v[pl.ds(192, 16)] = zi
    m_v[pl.ds(0, 16)] = zf
    for c in range(UROW // 16):
        u_rows0[0, pl.ds(c * 16, 16)] = zf
        u_rows1[0, pl.ds(c * 16, 16)] = zf
        u_rows0[LP + 1, pl.ds(c * 16, 16)] = zf
        u_rows1[LP + 1, pl.ds(c * 16, 16)] = zf

    bufs = ((idx_a0, idx_b0, w_rows0, u_rows0, sem0),
            (idx_a1, idx_b1, w_rows1, u_rows1, sem1))

    _load_seq_and_indices(seq_hbm, base, seq_v, idx_a0, idx_b0)
    _fire_gathers(w_hbm, u3_hbm, idx_a0, idx_b0, w_rows0, u_rows0, sem0)

    @pl.loop(0, RPW, step=2)
    def _pair(j):
        for b in range(2):
            r = j + b
            row = base + r
            ia, ib, wr_, ur, sem = bufs[b]
            ia2, ib2, wr2, ur2, sem2 = bufs[1 - b]

            _compute_masks(seq_v, m_v)

            @pl.when(r + 1 < RPW)
            def _prefetch():
                _load_seq_and_indices(seq_hbm, row + 1, seq_v, ia2, ib2)
                _fire_gathers(w_hbm, u3_hbm, ia2, ib2, wr2, ur2, sem2)

            _wait_gathers(w_hbm, u3_hbm, ia, ib, wr_, ur, sem)

            @pl.when(r > 0)
            def _drain_prev_out():
                pltpu.make_async_copy(
                    h_rows.at[pl.ds(0, L * EMB)],
                    out_hbm.at[pl.ds((row - 1) * L * EMB, L * EMB)],
                    sem_out).wait()

            _compute_row(wr_, ur, m_v, h_rows)
            pltpu.async_copy(
                h_rows.at[pl.ds(0, L * EMB)],
                out_hbm.at[pl.ds(row * L * EMB, L * EMB)], sem_out)

    pltpu.make_async_copy(
        h_rows.at[pl.ds(0, L * EMB)],
        out_hbm.at[pl.ds((base + RPW - 1) * L * EMB, L * EMB)],
        sem_out).wait()


@jax.jit
def _run(seq, W, U3):
    mesh = plsc.VectorSubcoreMesh(
        core_axis_name="c", subcore_axis_name="s",
        num_cores=NC, num_subcores=NS)
    dbuf = [
        pltpu.VMEM((SPLIT,), jnp.int32),
        pltpu.VMEM((LP - SPLIT,), jnp.int32),
        pltpu.VMEM((LP, EMB), jnp.float32),
        pltpu.VMEM((LP + 2, UROW), jnp.float32),
    ]
    kfn = pl.kernel(
        _region_kernel,
        out_type=jax.ShapeDtypeStruct((B * L * EMB,), jnp.float32),
        mesh=mesh,
        compiler_params=pltpu.CompilerParams(
            use_tc_tiling_on_sc=False, needs_layout_passes=False),
        scratch_types=[
            pltpu.VMEM((LP,), jnp.int32),
            pltpu.VMEM((LP + 32,), jnp.float32),
            *dbuf, *dbuf,
            pltpu.VMEM((LP * EMB,), jnp.float32),
            pltpu.SemaphoreType.DMA,
            pltpu.SemaphoreType.DMA,
            pltpu.SemaphoreType.DMA,
        ],
    )
    return kfn(seq, W, U3)


def kernel(seq, W, U):
    seq = seq.astype(jnp.int32).reshape(B * L)
    U3 = U.reshape(VOCAB - 1, REGION * EMB)
    return _run(seq, W, U3).reshape(B, L, EMB)


# speedup vs baseline: 1.1883x; 1.0010x over previous
import jax
import jax.numpy as jnp
from jax import lax
from jax.experimental import pallas as pl
from jax.experimental.pallas import tpu as pltpu
from jax.experimental.pallas import tpu_sc as plsc

VOCAB = 100000
EMB = 64
REGION = 3
B, L = 1024, 200
UROW = REGION * EMB

NC, NS = 2, 16
NW = NC * NS
RPW = B // NW
LP = 208
SPLIT = 112
NCHUNK = LP // 16


def _fire_seq(seq_hbm, row, seq_v, sem_seq):
    pltpu.async_copy(
        seq_hbm.at[pl.ds(row * L, L)], seq_v.at[pl.ds(0, L)], sem_seq)


def _wait_seq(seq_hbm, row, seq_v, sem_seq):
    pltpu.make_async_copy(
        seq_hbm.at[pl.ds(row * L, L)], seq_v.at[pl.ds(0, L)], sem_seq).wait()


def _compute_indices(seq_v, idx_a, idx_b):
    for k in range(NCHUNK):
        s = seq_v[pl.ds(k * 16, 16)]
        idx = jnp.maximum(s, 1) - 1
        if k * 16 < SPLIT:
            idx_a[pl.ds(k * 16, 16)] = idx
        else:
            idx_b[pl.ds(k * 16 - SPLIT, 16)] = idx


def _compute_masks(seq_v, m_v):
    for k in range(NCHUNK):
        s = seq_v[pl.ds(k * 16, 16)]
        m_v[pl.ds(k * 16 + 16, 16)] = jnp.where(
            s != 0, jnp.float32(1.0), jnp.float32(0.0))


def _fire_gathers(w_hbm, u3_hbm, idx_a, idx_b, w_rows, u_rows, sem):
    pltpu.async_copy(w_hbm.at[idx_a], w_rows.at[pl.ds(0, SPLIT)], sem)
    pltpu.async_copy(w_hbm.at[idx_b],
                     w_rows.at[pl.ds(SPLIT, LP - SPLIT)], sem)
    pltpu.async_copy(u3_hbm.at[idx_a], u_rows.at[pl.ds(1, SPLIT)], sem)
    pltpu.async_copy(u3_hbm.at[idx_b],
                     u_rows.at[pl.ds(1 + SPLIT, LP - SPLIT)], sem)


def _wait_gathers(w_hbm, u3_hbm, idx_a, idx_b, w_rows, u_rows, sem):
    pltpu.make_async_copy(
        w_hbm.at[idx_a], w_rows.at[pl.ds(0, SPLIT)], sem).wait()
    pltpu.make_async_copy(
        w_hbm.at[idx_b], w_rows.at[pl.ds(SPLIT, LP - SPLIT)], sem).wait()
    pltpu.make_async_copy(
        u3_hbm.at[idx_a], u_rows.at[pl.ds(1, SPLIT)], sem).wait()
    pltpu.make_async_copy(
        u3_hbm.at[idx_b], u_rows.at[pl.ds(1 + SPLIT, LP - SPLIT)], sem).wait()


def _compute_row(w_rows, u_rows, m_v, h_rows):
    @pl.loop(0, NCHUNK)
    def _group(g):
        ma = m_v[pl.ds(g * 16, 16)]
        mb = m_v[pl.ds(g * 16 + 16, 16)]
        mc_ = m_v[pl.ds(g * 16 + 32, 16)]
        gb = g * 16
        for t in range(16):
            ml = ma[15] if t == 0 else mb[t - 1]
            mm = mb[t]
            mr = mc_[0] if t == 15 else mb[t + 1]
            for c in range(EMB // 16):
                w = w_rows[gb + t, pl.ds(c * 16, 16)] * mm
                p0 = u_rows[gb + t, pl.ds(c * 16, 16)] * w * ml
                p1 = u_rows[gb + t + 1, pl.ds(EMB + c * 16, 16)] * w
                p2 = u_rows[gb + t + 2, pl.ds(2 * EMB + c * 16, 16)] * w * mr
                h_rows[pl.ds((gb + t) * EMB + c * 16, 16)] = jnp.maximum(
                    jnp.maximum(p0, p1), p2)


def _region_kernel(seq_hbm, w_hbm, u3_hbm, out_hbm,
                   seq_v0, seq_v1, m_v,
                   idx_a0, idx_b0, w_rows0, u_rows0,
                   idx_a1, idx_b1, w_rows1, u_rows1,
                   h_rows, sem0, sem1, sem_out, sem_seq):
    wid = lax.axis_index("s") * NC + lax.axis_index("c")
    base = wid * RPW

    zi = jnp.zeros((16,), jnp.int32)
    zf = jnp.zeros((16,), jnp.float32)
    seq_v0[pl.ds(192, 16)] = zi
    seq_v1[pl.ds(192, 16)] = zi
    m_v[pl.ds(0, 16)] = zf
    for c in range(UROW // 16):
        u_rows0[0, pl.ds(c * 16, 16)] = zf
        u_rows1[0, pl.ds(c * 16, 16)] = zf
        u_rows0[LP + 1, pl.ds(c * 16, 16)] = zf
        u_rows1[LP + 1, pl.ds(c * 16, 16)] = zf

    bufs = ((idx_a0, idx_b0, w_rows0, u_rows0, sem0, seq_v0),
            (idx_a1, idx_b1, w_rows1, u_rows1, sem1, seq_v1))

    _fire_seq(seq_hbm, base, seq_v0, sem_seq)
    _wait_seq(seq_hbm, base, seq_v0, sem_seq)
    _compute_indices(seq_v0, idx_a0, idx_b0)
    _fire_gathers(w_hbm, u3_hbm, idx_a0, idx_b0, w_rows0, u_rows0, sem0)
    _fire_seq(seq_hbm, base + 1, seq_v1, sem_seq)

    @pl.loop(0, RPW, step=2)
    def _pair(j):
        for b in range(2):
            r = j + b
            row = base + r
            ia, ib, wr_, ur, sem, sv = bufs[b]
            ia2, ib2, wr2, ur2, sem2, sv2 = bufs[1 - b]

            _compute_masks(sv, m_v)

            @pl.when(r + 1 < RPW)
            def _prefetch():
                _wait_seq(seq_hbm, row + 1, sv2, sem_seq)
                _compute_indices(sv2, ia2, ib2)
                _fire_gathers(w_hbm, u3_hbm, ia2, ib2, wr2, ur2, sem2)

            @pl.when(r + 2 < RPW)
            def _prefetch_seq():
                _fire_seq(seq_hbm, row + 2, sv, sem_seq)

            _wait_gathers(w_hbm, u3_hbm, ia, ib, wr_, ur, sem)

            @pl.when(r > 0)
            def _drain_prev_out():
                pltpu.make_async_copy(
                    h_rows.at[pl.ds(0, L * EMB)],
                    out_hbm.at[pl.ds((row - 1) * L * EMB, L * EMB)],
                    sem_out).wait()

            _compute_row(wr_, ur, m_v, h_rows)
            pltpu.async_copy(
                h_rows.at[pl.ds(0, L * EMB)],
                out_hbm.at[pl.ds(row * L * EMB, L * EMB)], sem_out)

    pltpu.make_async_copy(
        h_rows.at[pl.ds(0, L * EMB)],
        out_hbm.at[pl.ds((base + RPW - 1) * L * EMB, L * EMB)],
        sem_out).wait()


@jax.jit
def _run(seq, W, U3):
    mesh = plsc.VectorSubcoreMesh(
        core_axis_name="c", subcore_axis_name="s",
        num_cores=NC, num_subcores=NS)
    dbuf = [
        pltpu.VMEM((SPLIT,), jnp.int32),
        pltpu.VMEM((LP - SPLIT,), jnp.int32),
        pltpu.VMEM((LP, EMB), jnp.float32),
        pltpu.VMEM((LP + 2, UROW), jnp.float32),
    ]
    kfn = pl.kernel(
        _region_kernel,
        out_type=jax.ShapeDtypeStruct((B * L * EMB,), jnp.float32),
        mesh=mesh,
        compiler_params=pltpu.CompilerParams(
            use_tc_tiling_on_sc=False, needs_layout_passes=False),
        scratch_types=[
            pltpu.VMEM((LP,), jnp.int32),
            pltpu.VMEM((LP,), jnp.int32),
            pltpu.VMEM((LP + 32,), jnp.float32),
            *dbuf, *dbuf,
            pltpu.VMEM((LP * EMB,), jnp.float32),
            pltpu.SemaphoreType.DMA,
            pltpu.SemaphoreType.DMA,
            pltpu.SemaphoreType.DMA,
            pltpu.SemaphoreType.DMA,
        ],
    )
    return kfn(seq, W, U3)


def kernel(seq, W, U):
    seq = seq.astype(jnp.int32).reshape(B * L)
    U3 = U.reshape(VOCAB - 1, REGION * EMB)
    return _run(seq, W, U3).reshape(B, L, EMB)
